# baseline (speedup 1.0000x reference)
import jax
import jax.numpy as jnp
from jax.experimental import pallas as pl
from jax.experimental.pallas import tpu as pltpu

_ROWS = 128
_N = 32768
_B = 16
_C = _ROWS // _B
_NBUF = 8
_PRIME = 6


def _compute_chunk(buf):
    x = buf[...]
    ssum = jnp.sum(x, axis=1, keepdims=True)
    mx = jnp.max(x, axis=1, keepdims=True)
    mn = jnp.min(x, axis=1, keepdims=True)
    f_last = 1.0 + (_N - 1) * mx - ssum
    pos = f_last > 0
    kz = jnp.where(pos, jnp.float32(_N - 1), jnp.float32(0.0))
    m_z = jnp.where(pos, ssum, mn)
    tau = (m_z + 1.0) / kz
    buf[...] = jnp.maximum(x - tau, 0.0)


def _body(z_hbm, o_hbm, *scratch):
    bufs = list(scratch[:_NBUF])
    si, so = scratch[_NBUF], scratch[_NBUF + 1]
    in_h = [None] * _C
    out_h = [None] * _C
    out_waited = [False] * _C
    for k in range(min(_PRIME, _C)):
        in_h[k] = pltpu.make_async_copy(
            z_hbm.at[pl.ds(k * _B, _B)], bufs[k % _NBUF], si.at[k % _NBUF])
        in_h[k].start()
    for k in range(_C):
        b = bufs[k % _NBUF]
        in_h[k].wait()
        _compute_chunk(b)
        out_h[k] = pltpu.make_async_copy(
            b, o_hbm.at[pl.ds(k * _B, _B)], so.at[k % _NBUF])
        out_h[k].start()
        nk = k + _PRIME
        if nk < _C:
            if nk - _NBUF >= 0:
                out_h[nk - _NBUF].wait()
                out_waited[nk - _NBUF] = True
            in_h[nk] = pltpu.make_async_copy(
                z_hbm.at[pl.ds(nk * _B, _B)], bufs[nk % _NBUF],
                si.at[nk % _NBUF])
            in_h[nk].start()
    for k in range(_C):
        if not out_waited[k]:
            out_h[k].wait()


def kernel(z):
    return pl.pallas_call(
        _body,
        in_specs=[pl.BlockSpec(memory_space=pl.ANY)],
        out_specs=pl.BlockSpec(memory_space=pl.ANY),
        out_shape=jax.ShapeDtypeStruct((_ROWS, _N), z.dtype),
        scratch_shapes=(
            [pltpu.VMEM((_B, _N), jnp.float32) for _ in range(_NBUF)]
            + [pltpu.SemaphoreType.DMA((_NBUF,)),
               pltpu.SemaphoreType.DMA((_NBUF,))]
        ),
    )(z)


# speedup vs baseline: 1.0371x; 1.0371x over previous
import jax
import jax.numpy as jnp
from jax.experimental import pallas as pl
from jax.experimental.pallas import tpu as pltpu

_ROWS = 128
_N = 32768
_B = 32
_C = _ROWS // _B
_NBUF = 4
_PRIME = 3


def _compute_chunk(buf):
    x = buf[...]
    ssum = jnp.sum(x, axis=1, keepdims=True)
    mx = jnp.max(x, axis=1, keepdims=True)
    mn = jnp.min(x, axis=1, keepdims=True)
    f_last = 1.0 + (_N - 1) * mx - ssum
    pos = f_last > 0
    kz = jnp.where(pos, jnp.float32(_N - 1), jnp.float32(0.0))
    m_z = jnp.where(pos, ssum, mn)
    tau = (m_z + 1.0) / kz
    buf[...] = jnp.maximum(x - tau, 0.0)


def _body(z_hbm, o_hbm, *scratch):
    bufs = list(scratch[:_NBUF])
    si, so = scratch[_NBUF], scratch[_NBUF + 1]
    in_h = [None] * _C
    out_h = [None] * _C
    out_waited = [False] * _C
    for k in range(min(_PRIME, _C)):
        in_h[k] = pltpu.make_async_copy(
            z_hbm.at[pl.ds(k * _B, _B)], bufs[k % _NBUF], si.at[k % _NBUF])
        in_h[k].start()
    for k in range(_C):
        b = bufs[k % _NBUF]
        in_h[k].wait()
        _compute_chunk(b)
        out_h[k] = pltpu.make_async_copy(
            b, o_hbm.at[pl.ds(k * _B, _B)], so.at[k % _NBUF])
        out_h[k].start()
        nk = k + _PRIME
        if nk < _C:
            if nk - _NBUF >= 0:
                out_h[nk - _NBUF].wait()
                out_waited[nk - _NBUF] = True
            in_h[nk] = pltpu.make_async_copy(
                z_hbm.at[pl.ds(nk * _B, _B)], bufs[nk % _NBUF],
                si.at[nk % _NBUF])
            in_h[nk].start()
    for k in range(_C):
        if not out_waited[k]:
            out_h[k].wait()


def kernel(z):
    return pl.pallas_call(
        _body,
        in_specs=[pl.BlockSpec(memory_space=pl.ANY)],
        out_specs=pl.BlockSpec(memory_space=pl.ANY),
        out_shape=jax.ShapeDtypeStruct((_ROWS, _N), z.dtype),
        scratch_shapes=(
            [pltpu.VMEM((_B, _N), jnp.float32) for _ in range(_NBUF)]
            + [pltpu.SemaphoreType.DMA((_NBUF,)),
               pltpu.SemaphoreType.DMA((_NBUF,))]
        ),
    )(z)
